# baseline (speedup 1.0000x reference)
import functools

import jax
import jax.numpy as jnp
from jax import lax
from jax.experimental import pallas as pl
from jax.experimental.pallas import tpu as pltpu
from jax.experimental.pallas import tpu_sc as plsc

_FEAT = 64
_BATCH = 16384
_CLASSES = 1000000
_ALPHA = 1.0
_EPS = 1e-12

_NC = 2
_NS = 16
_NW = _NC * _NS
_BPW = _BATCH // _NW
_GROUPS = _BPW // 16
_BLK = 16
_ROWS = _CLASSES // _BLK
_GROW = 16 * _FEAT
_ICHUNK = 128


def _rsqrt16(x):
    y = lax.bitcast_convert_type(x, jnp.int32)
    y = jnp.int32(0x5F3759DF) - (y >> 1)
    r = lax.bitcast_convert_type(y, jnp.float32)
    for _ in range(3):
        r = r * (1.5 - 0.5 * x * r * r)
    return r


def _body(featsT_hbm, labels_hbm, cflat_hbm, out_hbm, lab_v, f_v, idx_v,
          c_blk, acc_v, sem, fsem):
    wid = lax.axis_index("s") * _NC + lax.axis_index("c")
    base = wid * _BPW

    pltpu.sync_copy(labels_hbm.at[wid], lab_v)

    feats_cp = pltpu.async_copy(
        featsT_hbm.at[:, pl.ds(base, _BPW)], f_v, fsem)
    feats_cp.wait()

    zero16 = jnp.zeros((16,), jnp.float32)
    eps2 = jnp.float32(_EPS * _EPS)
    lane_iota = lax.iota(jnp.int32, 16)

    def group_body(g, loss_acc):
        col0 = g * 16
        lab16 = lab_v[pl.ds(col0, 16)]
        base16 = lab16 >> 4

        for f in range(_FEAT):
            idx_v[f // 8, pl.ds((f % 8) * 16, 16)] = (
                base16 + jnp.int32(f * _ROWS))

        gathers = [
            pltpu.async_copy(
                cflat_hbm.at[idx_v.at[k]],
                c_blk.at[pl.ds(k * _ICHUNK, _ICHUNK)],
                sem,
            )
            for k in range(_GROW // _ICHUNK)
        ]
        for cp in gathers:
            cp.wait()

        lane16 = lab16 & jnp.int32(_BLK - 1)

        def feat_body(f, carry):
            sf, sc, sfc = carry
            fv = f_v[f, pl.ds(col0, 16)]
            rows = lane_iota + jnp.int32(f * 16)
            cv = plsc.load_gather(c_blk, [rows, lane16])
            return sf + fv * fv, sc + cv * cv, sfc + fv * cv

        sf, sc, sfc = lax.fori_loop(
            0, _FEAT, feat_body, (zero16, zero16, zero16))

        mf = jnp.maximum(sf, eps2)
        mc = jnp.maximum(sc, eps2)
        p = jnp.maximum(mf * mc, jnp.float32(1e-34))
        loss16 = sf / mf + sc / mc - 2.0 * (sfc * _rsqrt16(p))
        return loss_acc + loss16

    acc_v[...] = lax.fori_loop(0, _GROUPS, group_body, zero16)
    pltpu.sync_copy(acc_v, out_hbm.at[wid])


@jax.jit
def kernel(feats, labels, centers):
    lab = labels.astype(jnp.int32).reshape(_NW, _BPW)
    cflat = centers.T.reshape(_FEAT * _ROWS, _BLK)
    mesh = plsc.VectorSubcoreMesh(core_axis_name="c", subcore_axis_name="s")
    run = functools.partial(
        pl.kernel,
        mesh=mesh,
        compiler_params=pltpu.CompilerParams(
            needs_layout_passes=False, use_tc_tiling_on_sc=False),
        out_type=jax.ShapeDtypeStruct((_NW, 16), jnp.float32),
        scratch_types=[
            pltpu.VMEM((_BPW,), jnp.int32),
            pltpu.VMEM((_FEAT, _BPW), jnp.float32),
            pltpu.VMEM((_GROW // _ICHUNK, _ICHUNK), jnp.int32),
            pltpu.VMEM((_GROW, _BLK), jnp.float32),
            pltpu.VMEM((16,), jnp.float32),
            pltpu.SemaphoreType.DMA,
            pltpu.SemaphoreType.DMA,
        ],
    )(_body)
    partial_losses = run(feats.T, lab, cflat)
    return _ALPHA * (jnp.sum(partial_losses) / _BATCH)


# speedup vs baseline: 7.9573x; 7.9573x over previous
import functools

import jax
import jax.numpy as jnp
from jax import lax
from jax.experimental import pallas as pl
from jax.experimental.pallas import tpu as pltpu
from jax.experimental.pallas import tpu_sc as plsc

_FEAT = 64
_BATCH = 16384
_ALPHA = 1.0
_EPS = 1e-12

_NC = 2
_NS = 16
_NW = _NC * _NS
_BPW = _BATCH // _NW
_GCHUNK = 128
_NCHUNK = _BPW // _GCHUNK

_TCB = 2048
_TCG = _BATCH // _TCB


def _gather_body(labels_hbm, centers_hbm, out_hbm, idx_v, c_v, sem):
    wid = lax.axis_index("s") * _NC + lax.axis_index("c")
    pltpu.sync_copy(labels_hbm.at[wid], idx_v)
    gathers = [
        pltpu.async_copy(
            centers_hbm.at[idx_v.at[k]],
            c_v.at[pl.ds(k * _GCHUNK, _GCHUNK)],
            sem,
        )
        for k in range(_NCHUNK)
    ]
    for g in gathers:
        g.wait()
    pltpu.sync_copy(c_v, out_hbm.at[pl.ds(wid * _BPW, _BPW)])


def _loss_body(f_ref, c_ref, o_ref):
    f = f_ref[...]
    c = c_ref[...]
    sf = jnp.sum(f * f, axis=1)
    sc = jnp.sum(c * c, axis=1)
    sfc = jnp.sum(f * c, axis=1)
    eps2 = jnp.float32(_EPS * _EPS)
    mf = jnp.maximum(sf, eps2)
    mc = jnp.maximum(sc, eps2)
    loss = sf / mf + sc / mc - 2.0 * sfc * lax.rsqrt(mf * mc)
    o_ref[...] = jnp.full((8, 128), jnp.sum(loss), jnp.float32)


@jax.jit
def kernel(feats, labels, centers):
    lab = labels.astype(jnp.int32).reshape(_NW, _NCHUNK, _GCHUNK)
    mesh = plsc.VectorSubcoreMesh(core_axis_name="c", subcore_axis_name="s")
    gather = functools.partial(
        pl.kernel,
        mesh=mesh,
        compiler_params=pltpu.CompilerParams(
            needs_layout_passes=False, use_tc_tiling_on_sc=False),
        out_type=jax.ShapeDtypeStruct((_BATCH, _FEAT), jnp.float32),
        scratch_types=[
            pltpu.VMEM((_NCHUNK, _GCHUNK), jnp.int32),
            pltpu.VMEM((_BPW, _FEAT), jnp.float32),
            pltpu.SemaphoreType.DMA,
        ],
    )(_gather_body)
    gathered = gather(lab, centers)

    total = pl.pallas_call(
        _loss_body,
        out_shape=jax.ShapeDtypeStruct((8, 128), jnp.float32),
    )(feats, gathered)
    return _ALPHA * (total[0, 0] / _BATCH)
